# Initial kernel scaffold; baseline (speedup 1.0000x reference)
import jax
import jax.numpy as jnp
from jax.experimental import pallas as pl


def kernel(x, embedding):
    raise NotImplementedError("write your pallas kernel here")


# speedup vs baseline: 1.6826x; 1.6826x over previous
import functools

import jax
import jax.numpy as jnp
from jax import lax
from jax.experimental import pallas as pl
from jax.experimental.pallas import tpu as pltpu
from jax.experimental.pallas import tpu_sc as plsc

VOCAB = 1000000
DIM = 64
B = 16384
L = 50

NC = 2
NS = 16
NW = NC * NS

CHUNK = 128
TOTAL = B * L
NCHUNKS = TOTAL // CHUNK
CPW = NCHUNKS // NW


def _gather_kernel(idx_hbm, table_hbm, out_hbm, idx_v, rows_v, sem):
    wid = lax.axis_index("c") * NS + lax.axis_index("s")
    pltpu.sync_copy(idx_hbm.at[pl.ds(wid * CPW, CPW)], idx_v)

    def body(g, carry):
        pltpu.async_copy(table_hbm.at[idx_v.at[g]], rows_v, sem).wait()
        base = (wid * CPW + g) * CHUNK
        pltpu.sync_copy(rows_v, out_hbm.at[pl.ds(base, CHUNK)])
        return carry

    lax.fori_loop(0, CPW, body, 0)


@jax.jit
def _embedding_lookup(x_chunks, embedding):
    mesh = plsc.VectorSubcoreMesh(core_axis_name="c", subcore_axis_name="s")
    fn = functools.partial(
        pl.kernel,
        mesh=mesh,
        out_type=jax.ShapeDtypeStruct((TOTAL, DIM), jnp.float32),
        scratch_types=[
            pltpu.VMEM((CPW, CHUNK), jnp.int32),
            pltpu.VMEM((CHUNK, DIM), jnp.float32),
            pltpu.SemaphoreType.DMA,
        ],
        compiler_params=pltpu.CompilerParams(use_tc_tiling_on_sc=False),
    )(_gather_kernel)
    return fn(x_chunks, embedding)


def kernel(x, embedding):
    x_chunks = x.reshape(NCHUNKS, CHUNK)
    out = _embedding_lookup(x_chunks, embedding)
    return out.reshape(B, L, DIM)


# speedup vs baseline: 1.8769x; 1.1155x over previous
import functools

import jax
import jax.numpy as jnp
from jax import lax
from jax.experimental import pallas as pl
from jax.experimental.pallas import tpu as pltpu
from jax.experimental.pallas import tpu_sc as plsc

VOCAB = 1000000
DIM = 64
B = 16384
L = 50

NC = 2
NS = 16
NW = NC * NS

CHUNK = 128
TOTAL = B * L
NCHUNKS = TOTAL // CHUNK
CPW = NCHUNKS // NW


NBUF = 8
NR = CPW // NBUF


def _gather_kernel(idx_hbm, table_hbm, out_hbm, idx_v, *rest):
    bufs = rest[0:NBUF]
    gsem = rest[NBUF:2 * NBUF]
    osem = rest[2 * NBUF:3 * NBUF]
    wid = lax.axis_index("c") * NS + lax.axis_index("s")
    cbase = wid * CPW
    pltpu.sync_copy(idx_hbm.at[pl.ds(cbase, CPW)], idx_v)

    for j in range(NBUF - 1):
        pltpu.async_copy(table_hbm.at[idx_v.at[j]], bufs[j], gsem[j])

    def body(r, carry):
        for j in range(NBUF):
            g = r * NBUF + j
            bo = (j - 1) % NBUF

            def wait_out(bo=bo, g=g):
                pltpu.make_async_copy(
                    bufs[bo],
                    out_hbm.at[pl.ds((cbase + g - 1) * CHUNK, CHUNK)],
                    osem[bo]).wait()

            def fire_gather(bo=bo, g=g):
                pltpu.async_copy(
                    table_hbm.at[idx_v.at[g + NBUF - 1]], bufs[bo], gsem[bo])

            if j == 0:
                @pl.when(r > 0)
                def _():
                    wait_out()
                fire_gather()
            else:
                wait_out()

                @pl.when(r < NR - 1)
                def _():
                    fire_gather()

            pltpu.make_async_copy(
                table_hbm.at[idx_v.at[g]], bufs[j], gsem[j]).wait()
            pltpu.async_copy(
                bufs[j],
                out_hbm.at[pl.ds((cbase + g) * CHUNK, CHUNK)], osem[j])
        return carry

    lax.fori_loop(0, NR, body, 0)
    pltpu.make_async_copy(
        bufs[NBUF - 1],
        out_hbm.at[pl.ds((cbase + CPW - 1) * CHUNK, CHUNK)],
        osem[NBUF - 1]).wait()


@jax.jit
def _embedding_lookup(x_chunks, embedding):
    mesh = plsc.VectorSubcoreMesh(core_axis_name="c", subcore_axis_name="s")
    fn = functools.partial(
        pl.kernel,
        mesh=mesh,
        out_type=jax.ShapeDtypeStruct((TOTAL, DIM), jnp.float32),
        scratch_types=(
            [pltpu.VMEM((CPW, CHUNK), jnp.int32)]
            + [pltpu.VMEM((CHUNK, DIM), jnp.float32) for _ in range(NBUF)]
            + [pltpu.SemaphoreType.DMA for _ in range(2 * NBUF)]
        ),
        compiler_params=pltpu.CompilerParams(use_tc_tiling_on_sc=False),
    )(_gather_kernel)
    return fn(x_chunks, embedding)


def kernel(x, embedding):
    x_chunks = x.reshape(NCHUNKS, CHUNK)
    out = _embedding_lookup(x_chunks, embedding)
    return out.reshape(B, L, DIM)
